# baseline (speedup 1.0000x reference)
import functools

import jax
import jax.numpy as jnp
from jax import lax
from jax.experimental import pallas as pl
from jax.experimental.pallas import tpu as pltpu
from jax.experimental.pallas import tpu_sc as plsc

N_NODES = 50000
N_EDGES = 800000
N_FRAGS = 5000
N_MOLS = 512
MAX_FRAGS = 32
D_EDGE = 6
N_DEG = 6
H = 64
HH = 32
FP = 128
FPH = 64
NW = 32
EC = 128
N_ECHUNK = N_EDGES // EC
F_PAD = N_FRAGS + 8
NXROW = N_MOLS * MAX_FRAGS

_mesh = plsc.VectorSubcoreMesh(core_axis_name="c", subcore_axis_name="s")


def _wid():
    return lax.axis_index("s") * 2 + lax.axis_index("c")


def _tile_node_slice(s):
    return s * 3128


def _zero_and_writeout_nodes(z_hbm, acc, out_hbm, s, out_base, do_zero):
    @pl.when(s < 15)
    def _():
        if do_zero:
            pltpu.sync_copy(z_hbm, acc.at[pl.ds(s * 3128, 3128)])
        else:
            pltpu.sync_copy(acc.at[pl.ds(s * 3128, 3128)],
                            out_hbm.at[pl.ds(out_base + s * 3128, 3128)])
    @pl.when(s == 15)
    def _():
        if do_zero:
            pltpu.sync_copy(z_hbm.at[pl.ds(0, 3080)], acc.at[pl.ds(46920, 3080)])
        else:
            pltpu.sync_copy(acc.at[pl.ds(46920, 3080)],
                            out_hbm.at[pl.ds(out_base + 46920, 3080)])


def _sc_edge_attr(ea8_hbm, dst_hbm, z_hbm, out_hbm,
                  rb0, rb1, db0, db1, sem0, sem1, acc):
    c = lax.axis_index("c")
    s = lax.axis_index("s")
    _zero_and_writeout_nodes(z_hbm, acc, out_hbm, s, 0, True)
    plsc.subcore_barrier()
    base = (c * 16 + s) * 25000
    RB, DB, SM = (rb0, rb1), (db0, db1), (sem0, sem1)

    def issue(j, t):
        off = base + j * EC
        pltpu.sync_copy(dst_hbm.at[pl.ds(off, EC)], DB[t])
        pltpu.async_copy(ea8_hbm.at[pl.ds(off, EC)], RB[t], SM[t])

    def finish(t):
        pltpu.make_async_copy(ea8_hbm.at[pl.ds(0, EC)], RB[t], SM[t]).wait()
        pltpu.sync_copy(RB[t], acc.at[DB[t]], add=True)

    issue(0, 0)
    def body(jj, _):
        issue(2 * jj + 1, 1)
        finish(0)
        issue(2 * jj + 2, 0)
        finish(1)
        return 0
    lax.fori_loop(0, 97, body, 0)
    finish(0)
    off = base + 195 * EC
    pltpu.sync_copy(dst_hbm.at[pl.ds(off, 40)], db0.at[pl.ds(0, 40)])
    pltpu.sync_copy(ea8_hbm.at[pl.ds(off, 40)], rb0.at[pl.ds(0, 40)])
    pltpu.sync_copy(rb0.at[pl.ds(0, 40)], acc.at[db0.at[pl.ds(0, 40)]], add=True)
    plsc.subcore_barrier()
    _zero_and_writeout_nodes(z_hbm, acc, out_hbm, s, c * N_NODES, False)


def _edge_attr_call(ea8, dst):
    return pl.kernel(
        _sc_edge_attr,
        out_type=jax.ShapeDtypeStruct((2 * N_NODES, 8), jnp.float32),
        mesh=_mesh,
        compiler_params=pltpu.CompilerParams(use_tc_tiling_on_sc=False),
        scratch_types=[
            pltpu.VMEM((EC, 8), jnp.float32),
            pltpu.VMEM((EC, 8), jnp.float32),
            pltpu.VMEM((EC,), jnp.int32),
            pltpu.VMEM((EC,), jnp.int32),
            pltpu.SemaphoreType.DMA,
            pltpu.SemaphoreType.DMA,
            pltpu.VMEM_SHARED((N_NODES, 8), jnp.float32),
        ],
    )(ea8, dst, jnp.zeros((3128, 8), jnp.float32))


NCH_T = 391
NFULL = 390


def _sc_msg(hi_hbm, pk_hbm, z_hbm, out_hbm,
            rb0, rb1, rb2, rb3, pb0, pb1, pb2, pb3,
            sg0, sg1, sg2, sg3, ss0, ss1, ss2, ss3, acc):
    c = lax.axis_index("c")
    s = lax.axis_index("s")
    _zero_and_writeout_nodes(z_hbm, acc, out_hbm, s, 0, True)
    plsc.subcore_barrier()
    crow = (c * 16 + s) * NCH_T
    RB = (rb0, rb1, rb2, rb3)
    PB = (pb0, pb1, pb2, pb3)
    SG = (sg0, sg1, sg2, sg3)
    SS = (ss0, ss1, ss2, ss3)

    def issue(j, t):
        pltpu.sync_copy(pk_hbm.at[crow + j], PB[t])
        pltpu.async_copy(hi_hbm.at[PB[t].at[0]], RB[t], SG[t])

    def finish(t):
        pltpu.make_async_copy(hi_hbm.at[PB[t].at[0]], RB[t], SG[t]).wait()
        pltpu.async_copy(RB[t], acc.at[PB[t].at[1]], SS[t], add=True)

    def drain(t):
        pltpu.make_async_copy(RB[t], acc.at[PB[t].at[1]], SS[t]).wait()

    issue(0, 0)
    issue(1, 1)
    issue(2, 2)
    def body(jj, _):
        for t in range(4):
            j = 4 * jj + t
            finish(t)
            jn = j + 3
            tn = (t + 3) % 4
            @pl.when(jn < NCH_T)
            def _():
                @pl.when(jn >= 4)
                def _():
                    drain(tn)
                issue(jn, tn)
        return 0
    lax.fori_loop(0, 97, body, 0)
    finish(0)
    finish(1)
    finish(2)
    drain(0)
    drain(1)
    drain(2)
    drain(3)
    plsc.subcore_barrier()
    _zero_and_writeout_nodes(z_hbm, acc, out_hbm, s, c * N_NODES, False)


def _msg_call(hi, pk):
    return pl.kernel(
        _sc_msg,
        out_type=jax.ShapeDtypeStruct((2 * N_NODES, HH), jnp.float32),
        mesh=_mesh,
        compiler_params=pltpu.CompilerParams(use_tc_tiling_on_sc=False),
        scratch_types=[
            pltpu.VMEM((EC, HH), jnp.float32),
            pltpu.VMEM((EC, HH), jnp.float32),
            pltpu.VMEM((EC, HH), jnp.float32),
            pltpu.VMEM((EC, HH), jnp.float32),
            pltpu.VMEM((2, EC), jnp.int32),
            pltpu.VMEM((2, EC), jnp.int32),
            pltpu.VMEM((2, EC), jnp.int32),
            pltpu.VMEM((2, EC), jnp.int32),
            pltpu.SemaphoreType.DMA,
            pltpu.SemaphoreType.DMA,
            pltpu.SemaphoreType.DMA,
            pltpu.SemaphoreType.DMA,
            pltpu.SemaphoreType.DMA,
            pltpu.SemaphoreType.DMA,
            pltpu.SemaphoreType.DMA,
            pltpu.SemaphoreType.DMA,
            pltpu.VMEM_SHARED((N_NODES + 8, HH), jnp.float32),
        ],
    )(hi, pk, jnp.zeros((3128, HH), jnp.float32))


def _sc_fp(s0_hbm, s1_hbm, ba_hbm, z_hbm, out_hbm, rows_v, idx_v, acc):
    c = lax.axis_index("c")
    s = lax.axis_index("s")
    w = _wid()
    @pl.when(s < 15)
    def _():
        pltpu.sync_copy(z_hbm, acc.at[pl.ds(s * 320, 320)])
    @pl.when(s == 15)
    def _():
        pltpu.sync_copy(z_hbm.at[pl.ds(0, 208)], acc.at[pl.ds(4800, 208)])
    plsc.subcore_barrier()
    n_chunk = N_NODES // EC
    def body(k, _):
        g = k * 16 + s
        @pl.when(g < n_chunk)
        def _():
            r0 = g * EC
            @pl.when(c == 0)
            def _():
                pltpu.sync_copy(s0_hbm.at[pl.ds(r0, EC)], rows_v)
            @pl.when(c == 1)
            def _():
                pltpu.sync_copy(s1_hbm.at[pl.ds(r0, EC)], rows_v)
            pltpu.sync_copy(ba_hbm.at[pl.ds(r0, EC)], idx_v)
            pltpu.sync_copy(rows_v, acc.at[idx_v], add=True)
        return 0
    lax.fori_loop(0, (n_chunk + 15) // 16, body, 0)
    @pl.when(w == 0)
    def _():
        r0 = n_chunk * EC
        rem = N_NODES - r0
        @pl.when(c == 0)
        def _():
            pltpu.sync_copy(s0_hbm.at[pl.ds(r0, rem)], rows_v.at[pl.ds(0, rem)])
        pltpu.sync_copy(ba_hbm.at[pl.ds(r0, rem)], idx_v.at[pl.ds(0, rem)])
        pltpu.sync_copy(rows_v.at[pl.ds(0, rem)], acc.at[idx_v.at[pl.ds(0, rem)]], add=True)
    @pl.when(w == 1)
    def _():
        r0 = n_chunk * EC
        rem = N_NODES - r0
        pltpu.sync_copy(s1_hbm.at[pl.ds(r0, rem)], rows_v.at[pl.ds(0, rem)])
        pltpu.sync_copy(ba_hbm.at[pl.ds(r0, rem)], idx_v.at[pl.ds(0, rem)])
        pltpu.sync_copy(rows_v.at[pl.ds(0, rem)], acc.at[idx_v.at[pl.ds(0, rem)]], add=True)
    plsc.subcore_barrier()
    @pl.when(s < 15)
    def _():
        pltpu.sync_copy(acc.at[pl.ds(s * 320, 320)],
                        out_hbm.at[pl.ds(c * F_PAD + s * 320, 320)])
    @pl.when(s == 15)
    def _():
        pltpu.sync_copy(acc.at[pl.ds(4800, 208)],
                        out_hbm.at[pl.ds(c * F_PAD + 4800, 208)])


def _fp_call(s0, s1, ba):
    return pl.kernel(
        _sc_fp,
        out_type=jax.ShapeDtypeStruct((2 * F_PAD, FPH), jnp.float32),
        mesh=_mesh,
        compiler_params=pltpu.CompilerParams(use_tc_tiling_on_sc=False),
        scratch_types=[
            pltpu.VMEM((EC, FPH), jnp.float32),
            pltpu.VMEM((EC,), jnp.int32),
            pltpu.VMEM_SHARED((F_PAD, FPH), jnp.float32),
        ],
    )(s0, s1, ba, jnp.zeros((320, FPH), jnp.float32))


def _sc_dense(fp1_hbm, fp2_hbm, si_hbm, msk_hbm, x_hbm, m_hbm, rows_v, idx_v, mrow_v, sem):
    c = lax.axis_index("c")
    s = lax.axis_index("s")
    w = _wid()
    per_t = NXROW // 16
    def body(k, _):
        r0 = s * per_t + k * EC
        pltpu.sync_copy(si_hbm.at[pl.ds(r0, EC)], idx_v)
        def addc(j, _):
            idx_v[pl.ds(j * 16, 16)] = idx_v[pl.ds(j * 16, 16)] + c * F_PAD
            return 0
        lax.fori_loop(0, EC // 16, addc, 0)
        pltpu.async_copy(fp1_hbm.at[idx_v], rows_v, sem).wait()
        pltpu.async_copy(fp2_hbm.at[idx_v], rows_v, sem, add=True).wait()
        pltpu.sync_copy(rows_v, x_hbm.at[pl.ds(c * NXROW + r0, EC)])
        return 0
    lax.fori_loop(0, per_t // EC, body, 0)
    @pl.when(c == 0)
    def _():
        pltpu.sync_copy(msk_hbm.at[pl.ds(s * per_t, per_t)], mrow_v)
        pltpu.sync_copy(mrow_v, m_hbm.at[pl.ds(s * per_t, per_t)])


def _dense_call(fp1, fp2, si, msk):
    return pl.kernel(
        _sc_dense,
        out_type=(jax.ShapeDtypeStruct((2 * NXROW, FPH), jnp.float32),
                  jax.ShapeDtypeStruct((NXROW,), jnp.float32)),
        mesh=_mesh,
        compiler_params=pltpu.CompilerParams(use_tc_tiling_on_sc=False),
        scratch_types=[
            pltpu.VMEM((EC, FPH), jnp.float32),
            pltpu.VMEM((EC,), jnp.int32),
            pltpu.VMEM((NXROW // 16,), jnp.float32),
            pltpu.SemaphoreType.DMA,
        ],
    )(fp1, fp2, si, msk)


BR = 2000


def _tc_dense(h_ref, m_ref0, m_ref1, e_ref0, e_ref1, wbig_ref, b_ref, wout_ref,
              hout_ref, s0_ref, s1_ref):
    hin = h_ref[...]
    e = e_ref0[...] + e_ref1[...]
    degc = jnp.minimum(e[:, 6:7], float(N_DEG - 1))
    inp = jnp.concatenate([hin, m_ref0[...], m_ref1[...], e], axis=1)
    P = jax.lax.dot_general(inp.astype(jnp.bfloat16),
                            wbig_ref[...].astype(jnp.bfloat16),
                            (((1,), (0,)), ((), ())),
                            preferred_element_type=jnp.float32)
    acc = jnp.zeros((BR, H), jnp.float32)
    for d in range(N_DEG):
        acc = acc + jnp.where(degc == float(d), P[:, d * H:(d + 1) * H], 0.0)
    h = jnp.maximum(acc + b_ref[...], 0.0)
    hout_ref[...] = h
    t = jax.lax.dot_general(h.astype(jnp.bfloat16),
                            wout_ref[...].astype(jnp.bfloat16),
                            (((1,), (0,)), ((), ())),
                            preferred_element_type=jnp.float32)
    t = t - jnp.max(t, axis=1, keepdims=True)
    et = jnp.exp(t)
    sm = et / jnp.sum(et, axis=1, keepdims=True)
    s0_ref[...] = sm[:, :FPH]
    s1_ref[...] = sm[:, FPH:]


def _tc_dense_call(h64, msgh, msge, wbig, b, wout64):
    nb = N_NODES // BR
    return pl.pallas_call(
        _tc_dense,
        grid=(nb,),
        in_specs=[
            pl.BlockSpec((BR, H), lambda i: (i, 0)),
            pl.BlockSpec((BR, HH), lambda i: (i, 0)),
            pl.BlockSpec((BR, HH), lambda i: (i + nb, 0)),
            pl.BlockSpec((BR, 8), lambda i: (i, 0)),
            pl.BlockSpec((BR, 8), lambda i: (i + nb, 0)),
            pl.BlockSpec((136, N_DEG * H), lambda i: (0, 0)),
            pl.BlockSpec((1, H), lambda i: (0, 0)),
            pl.BlockSpec((H, FP), lambda i: (0, 0)),
        ],
        out_specs=[
            pl.BlockSpec((BR, H), lambda i: (i, 0)),
            pl.BlockSpec((BR, FPH), lambda i: (i, 0)),
            pl.BlockSpec((BR, FPH), lambda i: (i, 0)),
        ],
        out_shape=[
            jax.ShapeDtypeStruct((N_NODES, H), jnp.float32),
            jax.ShapeDtypeStruct((N_NODES, FPH), jnp.float32),
            jax.ShapeDtypeStruct((N_NODES, FPH), jnp.float32),
        ],
    )(h64, msgh, msgh, msge, msge, wbig, b, wout64)


def _pack_weights(W, d_h, Wout):
    Wh = jnp.pad(W[:, :d_h, :], ((0, 0), (0, H - d_h), (0, 0)))
    Wm = jnp.pad(W[:, d_h:2 * d_h, :], ((0, 0), (0, H - d_h), (0, 0)))
    We = jnp.pad(W[:, 2 * d_h:, :], ((0, 0), (0, 2), (0, 0)))
    Wcat = jnp.concatenate([Wh, Wm, We], axis=1)
    Wbig = jnp.transpose(Wcat, (1, 0, 2)).reshape(136, N_DEG * H)
    Wout64 = jnp.pad(Wout, ((0, H - Wout.shape[0]), (0, 0)))
    return Wbig, Wout64


def kernel(x, edge_attr, W1, b1, W2, b2, Wout1, Wout2, edge_index,
           batch_atom, batch_index_global):
    src = edge_index[0]
    dst = edge_index[1]

    ea8 = jnp.concatenate(
        [edge_attr, jnp.ones((N_EDGES, 1), jnp.float32),
         jnp.zeros((N_EDGES, 1), jnp.float32)], axis=1)
    idx_t = jnp.pad((src * 2).reshape(16, 50000), ((0, 0), (0, 48)))
    dpad = jnp.broadcast_to(N_NODES + (jnp.arange(48) % 8), (16, 48)).astype(jnp.int32)
    dst_t = jnp.concatenate([dst.reshape(16, 50000), dpad], axis=1)
    idx_t = idx_t.reshape(16, NCH_T, 1, EC)
    dst_t = dst_t.reshape(16, NCH_T, 1, EC)
    pk = jnp.stack([jnp.concatenate([idx_t, dst_t], axis=2),
                    jnp.concatenate([idx_t + 1, dst_t], axis=2)], axis=0)
    pk = pk.reshape(2 * 16 * NCH_T, 2, EC)
    x64 = jnp.pad(x, ((0, 0), (0, 2)))
    xi = x64.reshape(2 * N_NODES, HH)
    Wbig1, Wout64_1 = _pack_weights(W1, 62, Wout1)
    Wbig2, Wout64_2 = _pack_weights(W2, 64, Wout2)

    mol_ids = jnp.arange(N_MOLS, dtype=jnp.int32)
    eq = (batch_index_global[None, :] == mol_ids[:, None]).astype(jnp.int32)
    counts = jnp.sum(eq, axis=1)
    starts = jnp.cumsum(counts) - counts
    pp = jnp.arange(MAX_FRAGS, dtype=jnp.int32)[None, :]
    si = starts[:, None].astype(jnp.int32) + pp
    valid = pp < jnp.minimum(counts[:, None], MAX_FRAGS)
    flat = (mol_ids[:, None] * MAX_FRAGS + pp).reshape(-1)
    si = jnp.where(valid.reshape(-1), si.reshape(-1), N_FRAGS + (flat % 8)).astype(jnp.int32)
    msk = valid.reshape(-1).astype(jnp.float32)

    msge = _edge_attr_call(ea8, dst)
    msgh1 = _msg_call(xi, pk)
    h1, s0_1, s1_1 = _tc_dense_call(x64, msgh1, msge, Wbig1, b1.reshape(1, H), Wout64_1)
    fp1 = _fp_call(s0_1, s1_1, batch_atom)

    hi2 = h1.reshape(2 * N_NODES, HH)
    msgh2 = _msg_call(hi2, pk)
    h2, s0_2, s1_2 = _tc_dense_call(h1, msgh2, msge, Wbig2, b2.reshape(1, H), Wout64_2)
    fp2 = _fp_call(s0_2, s1_2, batch_atom)

    xs, mflat = _dense_call(fp1, fp2, si, msk)

    X = jnp.concatenate(
        [xs[:NXROW].reshape(N_MOLS, MAX_FRAGS, FPH),
         xs[NXROW:].reshape(N_MOLS, MAX_FRAGS, FPH)], axis=-1)
    m = mflat.reshape(N_MOLS, MAX_FRAGS)
    return X, m


# speedup vs baseline: 1.0242x; 1.0242x over previous
import functools

import jax
import jax.numpy as jnp
from jax import lax
from jax.experimental import pallas as pl
from jax.experimental.pallas import tpu as pltpu
from jax.experimental.pallas import tpu_sc as plsc

N_NODES = 50000
N_EDGES = 800000
N_FRAGS = 5000
N_MOLS = 512
MAX_FRAGS = 32
D_EDGE = 6
N_DEG = 6
H = 64
HH = 32
FP = 128
FPH = 64
NW = 32
EC = 128
N_ECHUNK = N_EDGES // EC
F_PAD = N_FRAGS + 8
NXROW = N_MOLS * MAX_FRAGS

_mesh = plsc.VectorSubcoreMesh(core_axis_name="c", subcore_axis_name="s")


def _wid():
    return lax.axis_index("s") * 2 + lax.axis_index("c")


def _tile_node_slice(s):
    return s * 3128


def _zero_and_writeout_nodes(z_hbm, acc, out_hbm, s, out_base, do_zero):
    @pl.when(s < 15)
    def _():
        if do_zero:
            pltpu.sync_copy(z_hbm, acc.at[pl.ds(s * 3128, 3128)])
        else:
            pltpu.sync_copy(acc.at[pl.ds(s * 3128, 3128)],
                            out_hbm.at[pl.ds(out_base + s * 3128, 3128)])
    @pl.when(s == 15)
    def _():
        if do_zero:
            pltpu.sync_copy(z_hbm.at[pl.ds(0, 3080)], acc.at[pl.ds(46920, 3080)])
        else:
            pltpu.sync_copy(acc.at[pl.ds(46920, 3080)],
                            out_hbm.at[pl.ds(out_base + 46920, 3080)])


def _sc_edge_attr(ea8_hbm, dst_hbm, z_hbm, out_hbm,
                  rb0, rb1, db0, db1, sem0, sem1, acc):
    c = lax.axis_index("c")
    s = lax.axis_index("s")
    _zero_and_writeout_nodes(z_hbm, acc, out_hbm, s, 0, True)
    plsc.subcore_barrier()
    base = (c * 16 + s) * 25000
    RB, DB, SM = (rb0, rb1), (db0, db1), (sem0, sem1)

    def issue(j, t):
        off = base + j * EC
        pltpu.sync_copy(dst_hbm.at[pl.ds(off, EC)], DB[t])
        pltpu.async_copy(ea8_hbm.at[pl.ds(off, EC)], RB[t], SM[t])

    def finish(t):
        pltpu.make_async_copy(ea8_hbm.at[pl.ds(0, EC)], RB[t], SM[t]).wait()
        pltpu.sync_copy(RB[t], acc.at[DB[t]], add=True)

    issue(0, 0)
    def body(jj, _):
        issue(2 * jj + 1, 1)
        finish(0)
        issue(2 * jj + 2, 0)
        finish(1)
        return 0
    lax.fori_loop(0, 97, body, 0)
    finish(0)
    off = base + 195 * EC
    pltpu.sync_copy(dst_hbm.at[pl.ds(off, 40)], db0.at[pl.ds(0, 40)])
    pltpu.sync_copy(ea8_hbm.at[pl.ds(off, 40)], rb0.at[pl.ds(0, 40)])
    pltpu.sync_copy(rb0.at[pl.ds(0, 40)], acc.at[db0.at[pl.ds(0, 40)]], add=True)
    plsc.subcore_barrier()
    _zero_and_writeout_nodes(z_hbm, acc, out_hbm, s, c * N_NODES, False)


def _edge_attr_call(ea8, dst):
    return pl.kernel(
        _sc_edge_attr,
        out_type=jax.ShapeDtypeStruct((2 * N_NODES, 8), jnp.float32),
        mesh=_mesh,
        compiler_params=pltpu.CompilerParams(use_tc_tiling_on_sc=False),
        scratch_types=[
            pltpu.VMEM((EC, 8), jnp.float32),
            pltpu.VMEM((EC, 8), jnp.float32),
            pltpu.VMEM((EC,), jnp.int32),
            pltpu.VMEM((EC,), jnp.int32),
            pltpu.SemaphoreType.DMA,
            pltpu.SemaphoreType.DMA,
            pltpu.VMEM_SHARED((N_NODES, 8), jnp.float32),
        ],
    )(ea8, dst, jnp.zeros((3128, 8), jnp.float32))


NCH_T = 391
NFULL = 390


def _sc_msg(hi_hbm, pk_hbm, z_hbm, out_hbm,
            rb0, rb1, rb2, rb3, pb0, pb1, pb2, pb3,
            sg0, sg1, sg2, sg3, ss0, ss1, ss2, ss3, acc):
    c = lax.axis_index("c")
    s = lax.axis_index("s")
    _zero_and_writeout_nodes(z_hbm, acc, out_hbm, s, 0, True)
    plsc.subcore_barrier()
    crow = (c * 16 + s) * NCH_T
    RB = (rb0, rb1, rb2, rb3)
    PB = (pb0, pb1, pb2, pb3)
    SG = (sg0, sg1, sg2, sg3)
    SS = (ss0, ss1, ss2, ss3)

    def issue(j, t):
        pltpu.sync_copy(pk_hbm.at[crow + j], PB[t])
        pltpu.async_copy(hi_hbm.at[PB[t].at[0]], RB[t], SG[t])

    def finish(t):
        pltpu.make_async_copy(hi_hbm.at[PB[t].at[0]], RB[t], SG[t]).wait()
        pltpu.async_copy(RB[t], acc.at[PB[t].at[1]], SS[t], add=True)

    def drain(t):
        pltpu.make_async_copy(RB[t], acc.at[PB[t].at[1]], SS[t]).wait()

    issue(0, 0)
    issue(1, 1)
    issue(2, 2)
    def body(jj, _):
        for t in range(4):
            j = 4 * jj + t
            finish(t)
            jn = j + 3
            tn = (t + 3) % 4
            @pl.when(jn < NCH_T)
            def _():
                @pl.when(jn >= 4)
                def _():
                    drain(tn)
                issue(jn, tn)
        return 0
    lax.fori_loop(0, 97, body, 0)
    finish(0)
    finish(1)
    finish(2)
    drain(0)
    drain(1)
    drain(2)
    drain(3)
    plsc.subcore_barrier()
    _zero_and_writeout_nodes(z_hbm, acc, out_hbm, s, c * N_NODES, False)


def _msg_call(hi, pk):
    return pl.kernel(
        _sc_msg,
        out_type=jax.ShapeDtypeStruct((2 * N_NODES, HH), jnp.float32),
        mesh=_mesh,
        compiler_params=pltpu.CompilerParams(use_tc_tiling_on_sc=False),
        scratch_types=[
            pltpu.VMEM((EC, HH), jnp.float32),
            pltpu.VMEM((EC, HH), jnp.float32),
            pltpu.VMEM((EC, HH), jnp.float32),
            pltpu.VMEM((EC, HH), jnp.float32),
            pltpu.VMEM((2, EC), jnp.int32),
            pltpu.VMEM((2, EC), jnp.int32),
            pltpu.VMEM((2, EC), jnp.int32),
            pltpu.VMEM((2, EC), jnp.int32),
            pltpu.SemaphoreType.DMA,
            pltpu.SemaphoreType.DMA,
            pltpu.SemaphoreType.DMA,
            pltpu.SemaphoreType.DMA,
            pltpu.SemaphoreType.DMA,
            pltpu.SemaphoreType.DMA,
            pltpu.SemaphoreType.DMA,
            pltpu.SemaphoreType.DMA,
            pltpu.VMEM_SHARED((N_NODES + 8, HH), jnp.float32),
        ],
    )(hi, pk, jnp.zeros((3128, HH), jnp.float32))


def _sc_fp(s0_hbm, s1_hbm, ba_hbm, z_hbm, out_hbm,
           rows_v, rows_v1, idx_v, idx_v1, sem0, sem1, acc):
    c = lax.axis_index("c")
    s = lax.axis_index("s")
    w = _wid()
    @pl.when(s < 15)
    def _():
        pltpu.sync_copy(z_hbm, acc.at[pl.ds(s * 320, 320)])
    @pl.when(s == 15)
    def _():
        pltpu.sync_copy(z_hbm.at[pl.ds(0, 208)], acc.at[pl.ds(4800, 208)])
    plsc.subcore_barrier()
    n_chunk = N_NODES // EC
    RB, IB, SM = (rows_v, rows_v1), (idx_v, idx_v1), (sem0, sem1)

    def issue(k, t):
        g = k * 16 + s
        @pl.when(g < n_chunk)
        def _():
            r0 = g * EC
            pltpu.sync_copy(ba_hbm.at[pl.ds(r0, EC)], IB[t])
            @pl.when(c == 0)
            def _():
                pltpu.async_copy(s0_hbm.at[pl.ds(r0, EC)], RB[t], SM[t])
            @pl.when(c == 1)
            def _():
                pltpu.async_copy(s1_hbm.at[pl.ds(r0, EC)], RB[t], SM[t])

    def finish(k, t):
        g = k * 16 + s
        @pl.when(g < n_chunk)
        def _():
            pltpu.make_async_copy(s0_hbm.at[pl.ds(0, EC)], RB[t], SM[t]).wait()
            pltpu.sync_copy(RB[t], acc.at[IB[t]], add=True)

    issue(0, 0)
    def body(kk, _):
        issue(2 * kk + 1, 1)
        finish(2 * kk, 0)
        issue(2 * kk + 2, 0)
        finish(2 * kk + 1, 1)
        return 0
    lax.fori_loop(0, 13, body, 0)
    @pl.when(w == 0)
    def _():
        r0 = n_chunk * EC
        rem = N_NODES - r0
        @pl.when(c == 0)
        def _():
            pltpu.sync_copy(s0_hbm.at[pl.ds(r0, rem)], rows_v.at[pl.ds(0, rem)])
        pltpu.sync_copy(ba_hbm.at[pl.ds(r0, rem)], idx_v.at[pl.ds(0, rem)])
        pltpu.sync_copy(rows_v.at[pl.ds(0, rem)], acc.at[idx_v.at[pl.ds(0, rem)]], add=True)
    @pl.when(w == 1)
    def _():
        r0 = n_chunk * EC
        rem = N_NODES - r0
        pltpu.sync_copy(s1_hbm.at[pl.ds(r0, rem)], rows_v.at[pl.ds(0, rem)])
        pltpu.sync_copy(ba_hbm.at[pl.ds(r0, rem)], idx_v.at[pl.ds(0, rem)])
        pltpu.sync_copy(rows_v.at[pl.ds(0, rem)], acc.at[idx_v.at[pl.ds(0, rem)]], add=True)
    plsc.subcore_barrier()
    @pl.when(s < 15)
    def _():
        pltpu.sync_copy(acc.at[pl.ds(s * 320, 320)],
                        out_hbm.at[pl.ds(c * F_PAD + s * 320, 320)])
    @pl.when(s == 15)
    def _():
        pltpu.sync_copy(acc.at[pl.ds(4800, 208)],
                        out_hbm.at[pl.ds(c * F_PAD + 4800, 208)])


def _fp_call(s0, s1, ba):
    return pl.kernel(
        _sc_fp,
        out_type=jax.ShapeDtypeStruct((2 * F_PAD, FPH), jnp.float32),
        mesh=_mesh,
        compiler_params=pltpu.CompilerParams(use_tc_tiling_on_sc=False),
        scratch_types=[
            pltpu.VMEM((EC, FPH), jnp.float32),
            pltpu.VMEM((EC, FPH), jnp.float32),
            pltpu.VMEM((EC,), jnp.int32),
            pltpu.VMEM((EC,), jnp.int32),
            pltpu.SemaphoreType.DMA,
            pltpu.SemaphoreType.DMA,
            pltpu.VMEM_SHARED((F_PAD, FPH), jnp.float32),
        ],
    )(s0, s1, ba, jnp.zeros((320, FPH), jnp.float32))


def _sc_dense(fp1_hbm, fp2_hbm, si_hbm, msk_hbm, x_hbm, m_hbm, rows_v, idx_v, mrow_v, sem):
    c = lax.axis_index("c")
    s = lax.axis_index("s")
    w = _wid()
    per_t = NXROW // 16
    def body(k, _):
        r0 = s * per_t + k * EC
        pltpu.sync_copy(si_hbm.at[pl.ds(r0, EC)], idx_v)
        def addc(j, _):
            idx_v[pl.ds(j * 16, 16)] = idx_v[pl.ds(j * 16, 16)] + c * F_PAD
            return 0
        lax.fori_loop(0, EC // 16, addc, 0)
        pltpu.async_copy(fp1_hbm.at[idx_v], rows_v, sem).wait()
        pltpu.async_copy(fp2_hbm.at[idx_v], rows_v, sem, add=True).wait()
        pltpu.sync_copy(rows_v, x_hbm.at[pl.ds(c * NXROW + r0, EC)])
        return 0
    lax.fori_loop(0, per_t // EC, body, 0)
    @pl.when(c == 0)
    def _():
        pltpu.sync_copy(msk_hbm.at[pl.ds(s * per_t, per_t)], mrow_v)
        pltpu.sync_copy(mrow_v, m_hbm.at[pl.ds(s * per_t, per_t)])


def _dense_call(fp1, fp2, si, msk):
    return pl.kernel(
        _sc_dense,
        out_type=(jax.ShapeDtypeStruct((2 * NXROW, FPH), jnp.float32),
                  jax.ShapeDtypeStruct((NXROW,), jnp.float32)),
        mesh=_mesh,
        compiler_params=pltpu.CompilerParams(use_tc_tiling_on_sc=False),
        scratch_types=[
            pltpu.VMEM((EC, FPH), jnp.float32),
            pltpu.VMEM((EC,), jnp.int32),
            pltpu.VMEM((NXROW // 16,), jnp.float32),
            pltpu.SemaphoreType.DMA,
        ],
    )(fp1, fp2, si, msk)


BR = 2000


def _tc_dense(h_ref, m_ref0, m_ref1, e_ref0, e_ref1, wbig_ref, b_ref, wout_ref,
              hout_ref, s0_ref, s1_ref):
    hin = h_ref[...]
    e = e_ref0[...] + e_ref1[...]
    degc = jnp.minimum(e[:, 6:7], float(N_DEG - 1))
    inp = jnp.concatenate([hin, m_ref0[...], m_ref1[...], e], axis=1)
    P = jax.lax.dot_general(inp, wbig_ref[...], (((1,), (0,)), ((), ())),
                            preferred_element_type=jnp.float32)
    acc = jnp.zeros((BR, H), jnp.float32)
    for d in range(N_DEG):
        acc = acc + jnp.where(degc == float(d), P[:, d * H:(d + 1) * H], 0.0)
    h = jnp.maximum(acc + b_ref[...], 0.0)
    hout_ref[...] = h
    t = jax.lax.dot_general(h, wout_ref[...], (((1,), (0,)), ((), ())),
                            preferred_element_type=jnp.float32)
    t = t - jnp.max(t, axis=1, keepdims=True)
    et = jnp.exp(t)
    sm = et / jnp.sum(et, axis=1, keepdims=True)
    s0_ref[...] = sm[:, :FPH]
    s1_ref[...] = sm[:, FPH:]


def _tc_dense_call(h64, msgh, msge, wbig, b, wout64):
    nb = N_NODES // BR
    return pl.pallas_call(
        _tc_dense,
        grid=(nb,),
        in_specs=[
            pl.BlockSpec((BR, H), lambda i: (i, 0)),
            pl.BlockSpec((BR, HH), lambda i: (i, 0)),
            pl.BlockSpec((BR, HH), lambda i: (i + nb, 0)),
            pl.BlockSpec((BR, 8), lambda i: (i, 0)),
            pl.BlockSpec((BR, 8), lambda i: (i + nb, 0)),
            pl.BlockSpec((136, N_DEG * H), lambda i: (0, 0)),
            pl.BlockSpec((1, H), lambda i: (0, 0)),
            pl.BlockSpec((H, FP), lambda i: (0, 0)),
        ],
        out_specs=[
            pl.BlockSpec((BR, H), lambda i: (i, 0)),
            pl.BlockSpec((BR, FPH), lambda i: (i, 0)),
            pl.BlockSpec((BR, FPH), lambda i: (i, 0)),
        ],
        out_shape=[
            jax.ShapeDtypeStruct((N_NODES, H), jnp.float32),
            jax.ShapeDtypeStruct((N_NODES, FPH), jnp.float32),
            jax.ShapeDtypeStruct((N_NODES, FPH), jnp.float32),
        ],
    )(h64, msgh, msgh, msge, msge, wbig, b, wout64)


def _pack_weights(W, d_h, Wout):
    Wh = jnp.pad(W[:, :d_h, :], ((0, 0), (0, H - d_h), (0, 0)))
    Wm = jnp.pad(W[:, d_h:2 * d_h, :], ((0, 0), (0, H - d_h), (0, 0)))
    We = jnp.pad(W[:, 2 * d_h:, :], ((0, 0), (0, 2), (0, 0)))
    Wcat = jnp.concatenate([Wh, Wm, We], axis=1)
    Wbig = jnp.transpose(Wcat, (1, 0, 2)).reshape(136, N_DEG * H)
    Wout64 = jnp.pad(Wout, ((0, H - Wout.shape[0]), (0, 0)))
    return Wbig, Wout64


def kernel(x, edge_attr, W1, b1, W2, b2, Wout1, Wout2, edge_index,
           batch_atom, batch_index_global):
    src = edge_index[0]
    dst = edge_index[1]

    ea8 = jnp.concatenate(
        [edge_attr, jnp.ones((N_EDGES, 1), jnp.float32),
         jnp.zeros((N_EDGES, 1), jnp.float32)], axis=1)
    idx_t = jnp.pad((src * 2).reshape(16, 50000), ((0, 0), (0, 48)))
    dpad = jnp.broadcast_to(N_NODES + (jnp.arange(48) % 8), (16, 48)).astype(jnp.int32)
    dst_t = jnp.concatenate([dst.reshape(16, 50000), dpad], axis=1)
    idx_t = idx_t.reshape(16, NCH_T, 1, EC)
    dst_t = dst_t.reshape(16, NCH_T, 1, EC)
    pk = jnp.stack([jnp.concatenate([idx_t, dst_t], axis=2),
                    jnp.concatenate([idx_t + 1, dst_t], axis=2)], axis=0)
    pk = pk.reshape(2 * 16 * NCH_T, 2, EC)
    x64 = jnp.pad(x, ((0, 0), (0, 2)))
    xi = x64.reshape(2 * N_NODES, HH)
    Wbig1, Wout64_1 = _pack_weights(W1, 62, Wout1)
    Wbig2, Wout64_2 = _pack_weights(W2, 64, Wout2)

    mol_ids = jnp.arange(N_MOLS, dtype=jnp.int32)
    eq = (batch_index_global[None, :] == mol_ids[:, None]).astype(jnp.int32)
    counts = jnp.sum(eq, axis=1)
    starts = jnp.cumsum(counts) - counts
    pp = jnp.arange(MAX_FRAGS, dtype=jnp.int32)[None, :]
    si = starts[:, None].astype(jnp.int32) + pp
    valid = pp < jnp.minimum(counts[:, None], MAX_FRAGS)
    flat = (mol_ids[:, None] * MAX_FRAGS + pp).reshape(-1)
    si = jnp.where(valid.reshape(-1), si.reshape(-1), N_FRAGS + (flat % 8)).astype(jnp.int32)
    msk = valid.reshape(-1).astype(jnp.float32)

    msge = _edge_attr_call(ea8, dst)
    msgh1 = _msg_call(xi, pk)
    h1, s0_1, s1_1 = _tc_dense_call(x64, msgh1, msge, Wbig1, b1.reshape(1, H), Wout64_1)
    fp1 = _fp_call(s0_1, s1_1, batch_atom)

    hi2 = h1.reshape(2 * N_NODES, HH)
    msgh2 = _msg_call(hi2, pk)
    h2, s0_2, s1_2 = _tc_dense_call(h1, msgh2, msge, Wbig2, b2.reshape(1, H), Wout64_2)
    fp2 = _fp_call(s0_2, s1_2, batch_atom)

    xs, mflat = _dense_call(fp1, fp2, si, msk)

    X = jnp.concatenate(
        [xs[:NXROW].reshape(N_MOLS, MAX_FRAGS, FPH),
         xs[NXROW:].reshape(N_MOLS, MAX_FRAGS, FPH)], axis=-1)
    m = mflat.reshape(N_MOLS, MAX_FRAGS)
    return X, m


# speedup vs baseline: 1.0243x; 1.0000x over previous
import functools

import jax
import jax.numpy as jnp
from jax import lax
from jax.experimental import pallas as pl
from jax.experimental.pallas import tpu as pltpu
from jax.experimental.pallas import tpu_sc as plsc

N_NODES = 50000
N_EDGES = 800000
N_FRAGS = 5000
N_MOLS = 512
MAX_FRAGS = 32
D_EDGE = 6
N_DEG = 6
H = 64
HH = 32
FP = 128
FPH = 64
NW = 32
EC = 128
N_ECHUNK = N_EDGES // EC
F_PAD = N_FRAGS + 8
NXROW = N_MOLS * MAX_FRAGS

_mesh = plsc.VectorSubcoreMesh(core_axis_name="c", subcore_axis_name="s")


def _wid():
    return lax.axis_index("s") * 2 + lax.axis_index("c")


def _tile_node_slice(s):
    return s * 3128


def _zero_and_writeout_nodes(z_hbm, acc, out_hbm, s, out_base, do_zero):
    @pl.when(s < 15)
    def _():
        if do_zero:
            pltpu.sync_copy(z_hbm, acc.at[pl.ds(s * 3128, 3128)])
        else:
            pltpu.sync_copy(acc.at[pl.ds(s * 3128, 3128)],
                            out_hbm.at[pl.ds(out_base + s * 3128, 3128)])
    @pl.when(s == 15)
    def _():
        if do_zero:
            pltpu.sync_copy(z_hbm.at[pl.ds(0, 3080)], acc.at[pl.ds(46920, 3080)])
        else:
            pltpu.sync_copy(acc.at[pl.ds(46920, 3080)],
                            out_hbm.at[pl.ds(out_base + 46920, 3080)])


def _sc_edge_attr(ea8_hbm, dst_hbm, z_hbm, out_hbm,
                  rb0, rb1, db0, db1, sem0, sem1, acc):
    c = lax.axis_index("c")
    s = lax.axis_index("s")
    _zero_and_writeout_nodes(z_hbm, acc, out_hbm, s, 0, True)
    plsc.subcore_barrier()
    base = (c * 16 + s) * 25000
    RB, DB, SM = (rb0, rb1), (db0, db1), (sem0, sem1)

    def issue(j, t):
        off = base + j * EC
        pltpu.sync_copy(dst_hbm.at[pl.ds(off, EC)], DB[t])
        pltpu.async_copy(ea8_hbm.at[pl.ds(off, EC)], RB[t], SM[t])

    def finish(t):
        pltpu.make_async_copy(ea8_hbm.at[pl.ds(0, EC)], RB[t], SM[t]).wait()
        pltpu.sync_copy(RB[t], acc.at[DB[t]], add=True)

    issue(0, 0)
    def body(jj, _):
        issue(2 * jj + 1, 1)
        finish(0)
        issue(2 * jj + 2, 0)
        finish(1)
        return 0
    lax.fori_loop(0, 97, body, 0)
    finish(0)
    off = base + 195 * EC
    pltpu.sync_copy(dst_hbm.at[pl.ds(off, 40)], db0.at[pl.ds(0, 40)])
    pltpu.sync_copy(ea8_hbm.at[pl.ds(off, 40)], rb0.at[pl.ds(0, 40)])
    pltpu.sync_copy(rb0.at[pl.ds(0, 40)], acc.at[db0.at[pl.ds(0, 40)]], add=True)
    plsc.subcore_barrier()
    _zero_and_writeout_nodes(z_hbm, acc, out_hbm, s, c * N_NODES, False)


def _edge_attr_call(ea8, dst):
    return pl.kernel(
        _sc_edge_attr,
        out_type=jax.ShapeDtypeStruct((2 * N_NODES, 8), jnp.float32),
        mesh=_mesh,
        compiler_params=pltpu.CompilerParams(use_tc_tiling_on_sc=False),
        scratch_types=[
            pltpu.VMEM((EC, 8), jnp.float32),
            pltpu.VMEM((EC, 8), jnp.float32),
            pltpu.VMEM((EC,), jnp.int32),
            pltpu.VMEM((EC,), jnp.int32),
            pltpu.SemaphoreType.DMA,
            pltpu.SemaphoreType.DMA,
            pltpu.VMEM_SHARED((N_NODES, 8), jnp.float32),
        ],
    )(ea8, dst, jnp.zeros((3128, 8), jnp.float32))


NCH_T = 391
NFULL = 390


def _sc_msg(hi_hbm, pk_hbm, z_hbm, out_hbm,
            rb0, rb1, rb2, rb3, pb0, pb1, pb2, pb3,
            sg0, sg1, sg2, sg3, ss0, ss1, ss2, ss3, acc):
    c = lax.axis_index("c")
    s = lax.axis_index("s")
    _zero_and_writeout_nodes(z_hbm, acc, out_hbm, s, 0, True)
    plsc.subcore_barrier()
    crow = (c * 16 + s) * NCH_T
    RB = (rb0, rb1, rb2, rb3)
    PB = (pb0, pb1, pb2, pb3)
    SG = (sg0, sg1, sg2, sg3)
    SS = (ss0, ss1, ss2, ss3)

    def issue(j, t):
        pltpu.sync_copy(pk_hbm.at[crow + j], PB[t])
        pltpu.async_copy(hi_hbm.at[PB[t].at[0]], RB[t], SG[t])

    def finish(t):
        pltpu.make_async_copy(hi_hbm.at[PB[t].at[0]], RB[t], SG[t]).wait()
        pltpu.async_copy(RB[t], acc.at[PB[t].at[1]], SS[t], add=True)

    def drain(t):
        pltpu.make_async_copy(RB[t], acc.at[PB[t].at[1]], SS[t]).wait()

    issue(0, 0)
    issue(1, 1)
    issue(2, 2)
    def body(jj, _):
        for t in range(4):
            j = 4 * jj + t
            finish(t)
            jn = j + 3
            tn = (t + 3) % 4
            @pl.when(jn < NCH_T)
            def _():
                @pl.when(jn >= 4)
                def _():
                    drain(tn)
                issue(jn, tn)
        return 0
    lax.fori_loop(0, 97, body, 0)
    finish(0)
    finish(1)
    finish(2)
    drain(0)
    drain(1)
    drain(2)
    drain(3)
    plsc.subcore_barrier()
    _zero_and_writeout_nodes(z_hbm, acc, out_hbm, s, c * N_NODES, False)


def _msg_call(hi, pk):
    return pl.kernel(
        _sc_msg,
        out_type=jax.ShapeDtypeStruct((2 * N_NODES, HH), jnp.float32),
        mesh=_mesh,
        compiler_params=pltpu.CompilerParams(use_tc_tiling_on_sc=False),
        scratch_types=[
            pltpu.VMEM((EC, HH), jnp.float32),
            pltpu.VMEM((EC, HH), jnp.float32),
            pltpu.VMEM((EC, HH), jnp.float32),
            pltpu.VMEM((EC, HH), jnp.float32),
            pltpu.VMEM((2, EC), jnp.int32),
            pltpu.VMEM((2, EC), jnp.int32),
            pltpu.VMEM((2, EC), jnp.int32),
            pltpu.VMEM((2, EC), jnp.int32),
            pltpu.SemaphoreType.DMA,
            pltpu.SemaphoreType.DMA,
            pltpu.SemaphoreType.DMA,
            pltpu.SemaphoreType.DMA,
            pltpu.SemaphoreType.DMA,
            pltpu.SemaphoreType.DMA,
            pltpu.SemaphoreType.DMA,
            pltpu.SemaphoreType.DMA,
            pltpu.VMEM_SHARED((N_NODES + 8, HH), jnp.float32),
        ],
    )(hi, pk, jnp.zeros((3128, HH), jnp.float32))


def _sc_fp(s0_hbm, s1_hbm, ba_hbm, z_hbm, out_hbm,
           rows_v, rows_v1, idx_v, idx_v1, sem0, sem1, acc):
    c = lax.axis_index("c")
    s = lax.axis_index("s")
    w = _wid()
    @pl.when(s < 15)
    def _():
        pltpu.sync_copy(z_hbm, acc.at[pl.ds(s * 320, 320)])
    @pl.when(s == 15)
    def _():
        pltpu.sync_copy(z_hbm.at[pl.ds(0, 208)], acc.at[pl.ds(4800, 208)])
    plsc.subcore_barrier()
    n_chunk = N_NODES // EC
    RB, IB, SM = (rows_v, rows_v1), (idx_v, idx_v1), (sem0, sem1)

    def issue(k, t):
        g = k * 16 + s
        @pl.when(g < n_chunk)
        def _():
            r0 = g * EC
            pltpu.sync_copy(ba_hbm.at[pl.ds(r0, EC)], IB[t])
            @pl.when(c == 0)
            def _():
                pltpu.async_copy(s0_hbm.at[pl.ds(r0, EC)], RB[t], SM[t])
            @pl.when(c == 1)
            def _():
                pltpu.async_copy(s1_hbm.at[pl.ds(r0, EC)], RB[t], SM[t])

    def finish(k, t):
        g = k * 16 + s
        @pl.when(g < n_chunk)
        def _():
            pltpu.make_async_copy(s0_hbm.at[pl.ds(0, EC)], RB[t], SM[t]).wait()
            pltpu.sync_copy(RB[t], acc.at[IB[t]], add=True)

    issue(0, 0)
    def body(kk, _):
        issue(2 * kk + 1, 1)
        finish(2 * kk, 0)
        issue(2 * kk + 2, 0)
        finish(2 * kk + 1, 1)
        return 0
    lax.fori_loop(0, 13, body, 0)
    @pl.when(w == 0)
    def _():
        r0 = n_chunk * EC
        rem = N_NODES - r0
        @pl.when(c == 0)
        def _():
            pltpu.sync_copy(s0_hbm.at[pl.ds(r0, rem)], rows_v.at[pl.ds(0, rem)])
        pltpu.sync_copy(ba_hbm.at[pl.ds(r0, rem)], idx_v.at[pl.ds(0, rem)])
        pltpu.sync_copy(rows_v.at[pl.ds(0, rem)], acc.at[idx_v.at[pl.ds(0, rem)]], add=True)
    @pl.when(w == 1)
    def _():
        r0 = n_chunk * EC
        rem = N_NODES - r0
        pltpu.sync_copy(s1_hbm.at[pl.ds(r0, rem)], rows_v.at[pl.ds(0, rem)])
        pltpu.sync_copy(ba_hbm.at[pl.ds(r0, rem)], idx_v.at[pl.ds(0, rem)])
        pltpu.sync_copy(rows_v.at[pl.ds(0, rem)], acc.at[idx_v.at[pl.ds(0, rem)]], add=True)
    plsc.subcore_barrier()
    @pl.when(s < 15)
    def _():
        pltpu.sync_copy(acc.at[pl.ds(s * 320, 320)],
                        out_hbm.at[pl.ds(c * F_PAD + s * 320, 320)])
    @pl.when(s == 15)
    def _():
        pltpu.sync_copy(acc.at[pl.ds(4800, 208)],
                        out_hbm.at[pl.ds(c * F_PAD + 4800, 208)])


def _fp_call(s0, s1, ba):
    return pl.kernel(
        _sc_fp,
        out_type=jax.ShapeDtypeStruct((2 * F_PAD, FPH), jnp.float32),
        mesh=_mesh,
        compiler_params=pltpu.CompilerParams(use_tc_tiling_on_sc=False),
        scratch_types=[
            pltpu.VMEM((EC, FPH), jnp.float32),
            pltpu.VMEM((EC, FPH), jnp.float32),
            pltpu.VMEM((EC,), jnp.int32),
            pltpu.VMEM((EC,), jnp.int32),
            pltpu.SemaphoreType.DMA,
            pltpu.SemaphoreType.DMA,
            pltpu.VMEM_SHARED((F_PAD, FPH), jnp.float32),
        ],
    )(s0, s1, ba, jnp.zeros((320, FPH), jnp.float32))


def _sc_dense(fp1_hbm, fp2_hbm, si_hbm, msk_hbm, x_hbm, m_hbm,
              rv0, rv1, iv0, iv1, mrow_v, sem0, sem1):
    c = lax.axis_index("c")
    s = lax.axis_index("s")
    per_t = NXROW // 16
    nk = per_t // EC
    RB, IB, SM = (rv0, rv1), (iv0, iv1), (sem0, sem1)

    def issue(k):
        t = k % 2
        r0 = s * per_t + k * EC
        iv = IB[t]
        pltpu.sync_copy(si_hbm.at[pl.ds(r0, EC)], iv)
        def addc(j, _):
            iv[pl.ds(j * 16, 16)] = iv[pl.ds(j * 16, 16)] + c * F_PAD
            return 0
        lax.fori_loop(0, EC // 16, addc, 0)
        pltpu.async_copy(fp1_hbm.at[iv], RB[t], SM[t])

    def finish(k):
        t = k % 2
        r0 = s * per_t + k * EC
        pltpu.make_async_copy(fp1_hbm.at[IB[t]], RB[t], SM[t]).wait()
        pltpu.async_copy(fp2_hbm.at[IB[t]], RB[t], SM[t], add=True).wait()
        pltpu.sync_copy(RB[t], x_hbm.at[pl.ds(c * NXROW + r0, EC)])

    issue(0)
    for k in range(nk):
        if k + 1 < nk:
            issue(k + 1)
        finish(k)
    @pl.when(c == 0)
    def _():
        pltpu.sync_copy(msk_hbm.at[pl.ds(s * per_t, per_t)], mrow_v)
        pltpu.sync_copy(mrow_v, m_hbm.at[pl.ds(s * per_t, per_t)])


def _dense_call(fp1, fp2, si, msk):
    return pl.kernel(
        _sc_dense,
        out_type=(jax.ShapeDtypeStruct((2 * NXROW, FPH), jnp.float32),
                  jax.ShapeDtypeStruct((NXROW,), jnp.float32)),
        mesh=_mesh,
        compiler_params=pltpu.CompilerParams(use_tc_tiling_on_sc=False),
        scratch_types=[
            pltpu.VMEM((EC, FPH), jnp.float32),
            pltpu.VMEM((EC, FPH), jnp.float32),
            pltpu.VMEM((EC,), jnp.int32),
            pltpu.VMEM((EC,), jnp.int32),
            pltpu.VMEM((NXROW // 16,), jnp.float32),
            pltpu.SemaphoreType.DMA,
            pltpu.SemaphoreType.DMA,
        ],
    )(fp1, fp2, si, msk)


BR = 2000


def _tc_dense(h_ref, m_ref0, m_ref1, e_ref0, e_ref1, wbig_ref, b_ref, wout_ref,
              hout_ref, s0_ref, s1_ref):
    hin = h_ref[...]
    e = e_ref0[...] + e_ref1[...]
    degc = jnp.minimum(e[:, 6:7], float(N_DEG - 1))
    inp = jnp.concatenate([hin, m_ref0[...], m_ref1[...], e], axis=1)
    P = jax.lax.dot_general(inp, wbig_ref[...], (((1,), (0,)), ((), ())),
                            preferred_element_type=jnp.float32)
    acc = jnp.zeros((BR, H), jnp.float32)
    for d in range(N_DEG):
        acc = acc + jnp.where(degc == float(d), P[:, d * H:(d + 1) * H], 0.0)
    h = jnp.maximum(acc + b_ref[...], 0.0)
    hout_ref[...] = h
    t = jax.lax.dot_general(h, wout_ref[...], (((1,), (0,)), ((), ())),
                            preferred_element_type=jnp.float32)
    t = t - jnp.max(t, axis=1, keepdims=True)
    et = jnp.exp(t)
    sm = et / jnp.sum(et, axis=1, keepdims=True)
    s0_ref[...] = sm[:, :FPH]
    s1_ref[...] = sm[:, FPH:]


def _tc_dense_call(h64, msgh, msge, wbig, b, wout64):
    nb = N_NODES // BR
    return pl.pallas_call(
        _tc_dense,
        grid=(nb,),
        in_specs=[
            pl.BlockSpec((BR, H), lambda i: (i, 0)),
            pl.BlockSpec((BR, HH), lambda i: (i, 0)),
            pl.BlockSpec((BR, HH), lambda i: (i + nb, 0)),
            pl.BlockSpec((BR, 8), lambda i: (i, 0)),
            pl.BlockSpec((BR, 8), lambda i: (i + nb, 0)),
            pl.BlockSpec((136, N_DEG * H), lambda i: (0, 0)),
            pl.BlockSpec((1, H), lambda i: (0, 0)),
            pl.BlockSpec((H, FP), lambda i: (0, 0)),
        ],
        out_specs=[
            pl.BlockSpec((BR, H), lambda i: (i, 0)),
            pl.BlockSpec((BR, FPH), lambda i: (i, 0)),
            pl.BlockSpec((BR, FPH), lambda i: (i, 0)),
        ],
        out_shape=[
            jax.ShapeDtypeStruct((N_NODES, H), jnp.float32),
            jax.ShapeDtypeStruct((N_NODES, FPH), jnp.float32),
            jax.ShapeDtypeStruct((N_NODES, FPH), jnp.float32),
        ],
    )(h64, msgh, msgh, msge, msge, wbig, b, wout64)


def _pack_weights(W, d_h, Wout):
    Wh = jnp.pad(W[:, :d_h, :], ((0, 0), (0, H - d_h), (0, 0)))
    Wm = jnp.pad(W[:, d_h:2 * d_h, :], ((0, 0), (0, H - d_h), (0, 0)))
    We = jnp.pad(W[:, 2 * d_h:, :], ((0, 0), (0, 2), (0, 0)))
    Wcat = jnp.concatenate([Wh, Wm, We], axis=1)
    Wbig = jnp.transpose(Wcat, (1, 0, 2)).reshape(136, N_DEG * H)
    Wout64 = jnp.pad(Wout, ((0, H - Wout.shape[0]), (0, 0)))
    return Wbig, Wout64


def kernel(x, edge_attr, W1, b1, W2, b2, Wout1, Wout2, edge_index,
           batch_atom, batch_index_global):
    src = edge_index[0]
    dst = edge_index[1]

    ea8 = jnp.concatenate(
        [edge_attr, jnp.ones((N_EDGES, 1), jnp.float32),
         jnp.zeros((N_EDGES, 1), jnp.float32)], axis=1)
    idx_t = jnp.pad((src * 2).reshape(16, 50000), ((0, 0), (0, 48)))
    dpad = jnp.broadcast_to(N_NODES + (jnp.arange(48) % 8), (16, 48)).astype(jnp.int32)
    dst_t = jnp.concatenate([dst.reshape(16, 50000), dpad], axis=1)
    idx_t = idx_t.reshape(16, NCH_T, 1, EC)
    dst_t = dst_t.reshape(16, NCH_T, 1, EC)
    pk = jnp.stack([jnp.concatenate([idx_t, dst_t], axis=2),
                    jnp.concatenate([idx_t + 1, dst_t], axis=2)], axis=0)
    pk = pk.reshape(2 * 16 * NCH_T, 2, EC)
    x64 = jnp.pad(x, ((0, 0), (0, 2)))
    xi = x64.reshape(2 * N_NODES, HH)
    Wbig1, Wout64_1 = _pack_weights(W1, 62, Wout1)
    Wbig2, Wout64_2 = _pack_weights(W2, 64, Wout2)

    mol_ids = jnp.arange(N_MOLS, dtype=jnp.int32)
    eq = (batch_index_global[None, :] == mol_ids[:, None]).astype(jnp.int32)
    counts = jnp.sum(eq, axis=1)
    starts = jnp.cumsum(counts) - counts
    pp = jnp.arange(MAX_FRAGS, dtype=jnp.int32)[None, :]
    si = starts[:, None].astype(jnp.int32) + pp
    valid = pp < jnp.minimum(counts[:, None], MAX_FRAGS)
    flat = (mol_ids[:, None] * MAX_FRAGS + pp).reshape(-1)
    si = jnp.where(valid.reshape(-1), si.reshape(-1), N_FRAGS + (flat % 8)).astype(jnp.int32)
    msk = valid.reshape(-1).astype(jnp.float32)

    msge = _edge_attr_call(ea8, dst)
    msgh1 = _msg_call(xi, pk)
    h1, s0_1, s1_1 = _tc_dense_call(x64, msgh1, msge, Wbig1, b1.reshape(1, H), Wout64_1)
    fp1 = _fp_call(s0_1, s1_1, batch_atom)

    hi2 = h1.reshape(2 * N_NODES, HH)
    msgh2 = _msg_call(hi2, pk)
    h2, s0_2, s1_2 = _tc_dense_call(h1, msgh2, msge, Wbig2, b2.reshape(1, H), Wout64_2)
    fp2 = _fp_call(s0_2, s1_2, batch_atom)

    xs, mflat = _dense_call(fp1, fp2, si, msk)

    X = jnp.concatenate(
        [xs[:NXROW].reshape(N_MOLS, MAX_FRAGS, FPH),
         xs[NXROW:].reshape(N_MOLS, MAX_FRAGS, FPH)], axis=-1)
    m = mflat.reshape(N_MOLS, MAX_FRAGS)
    return X, m
